# baseline (speedup 1.0000x reference)
import functools

import jax
import jax.numpy as jnp
from jax import lax
from jax.experimental import pallas as pl
from jax.experimental.pallas import tpu as pltpu
from jax.experimental.pallas import tpu_sc as plsc

VOCAB = 100000
EMBED_DIM = 64
PADDED = 128
BATCH = 4096
HIST = 50

TOTAL = BATCH * HIST
NUM_CORES = 2
NUM_SUBCORES = 16
NW = NUM_CORES * NUM_SUBCORES
BPW = TOTAL // NW
CHUNK = 400
NCHUNK = BPW // CHUNK
NBUF = 4

_mesh = plsc.VectorSubcoreMesh(core_axis_name="c", subcore_axis_name="s")


@functools.partial(
    pl.kernel,
    out_type=jax.ShapeDtypeStruct((TOTAL, PADDED), jnp.float32),
    mesh=_mesh,
    scratch_types=[
        pltpu.VMEM((BPW,), jnp.int32),
        pltpu.VMEM((CHUNK, EMBED_DIM), jnp.float32),
        pltpu.VMEM((CHUNK, EMBED_DIM), jnp.float32),
        pltpu.VMEM((CHUNK, EMBED_DIM), jnp.float32),
        pltpu.VMEM((CHUNK, EMBED_DIM), jnp.float32),
        pltpu.SemaphoreType.DMA,
        pltpu.SemaphoreType.DMA,
        pltpu.SemaphoreType.DMA,
        pltpu.SemaphoreType.DMA,
        pltpu.SemaphoreType.DMA,
        pltpu.SemaphoreType.DMA,
        pltpu.SemaphoreType.DMA,
        pltpu.SemaphoreType.DMA,
    ],
    compiler_params=pltpu.CompilerParams(use_tc_tiling_on_sc=False),
)
def _embed_gather(table_hbm, idx_hbm, out_hbm, idx_v, rows0, rows1, rows2,
                  rows3, g0, g1, g2, g3, s0, s1, s2, s3):
    wid = lax.axis_index("s") * NUM_CORES + lax.axis_index("c")
    base = wid * BPW
    pltpu.sync_copy(idx_hbm.at[pl.ds(base, BPW)], idx_v)

    rows = (rows0, rows1, rows2, rows3)
    gsem = (g0, g1, g2, g3)
    ssem = (s0, s1, s2, s3)

    def gather(n, b):
        off = pl.multiple_of(n * CHUNK, CHUNK)
        pltpu.async_copy(table_hbm.at[idx_v.at[pl.ds(off, CHUNK)]],
                         rows[b], gsem[b])

    def store(n, b):
        off = pl.multiple_of(n * CHUNK, CHUNK)
        pltpu.async_copy(
            rows[b],
            out_hbm.at[pl.ds(base + off, CHUNK), pl.ds(0, EMBED_DIM)],
            ssem[b])

    def wait_gather(b):
        pltpu.make_async_copy(table_hbm.at[idx_v.at[pl.ds(0, CHUNK)]],
                              rows[b], gsem[b]).wait()

    def wait_store(b):
        pltpu.make_async_copy(
            rows[b],
            out_hbm.at[pl.ds(0, CHUNK), pl.ds(0, EMBED_DIM)],
            ssem[b]).wait()

    for b in range(NBUF):
        gather(b, b)

    def body(i, carry):
        for b in range(NBUF):
            n = NBUF * i + b
            wait_gather(b)
            store(n, b)

            @pl.when(n + NBUF < NCHUNK)
            def _():
                wait_store(b)
                gather(n + NBUF, b)

        return carry

    lax.fori_loop(0, NCHUNK // NBUF, body, 0)


def kernel(input_tensor, table):
    tpad = jnp.pad(table, ((0, 0), (0, PADDED - EMBED_DIM)))
    t2 = tpad.reshape(2 * VOCAB, EMBED_DIM)
    idxt = input_tensor.T.reshape(-1).astype(jnp.int32) * 2
    out = _embed_gather(t2, idxt)
    out = out.reshape(HIST, BATCH, PADDED)[:, :, :EMBED_DIM]
    return out.transpose(1, 0, 2)
